# baseline (speedup 1.0000x reference)
import functools

import numpy as np
import jax
import jax.numpy as jnp
from jax import lax
from jax.experimental import pallas as pl
from jax.experimental.pallas import tpu as pltpu
from jax.experimental.pallas import tpu_sc as plsc

N = 262144
NLEV = 16
T = 1 << 19
BASE_RES = 16
SCALE = 1.3819128799
P1 = np.int32(np.uint32(2654435761).astype(np.int32))
P2 = np.int32(805459861)
OUT_D = 3 + 2 * NLEV

_info = plsc.get_sparse_core_info()
NC = _info.num_cores
NW = _info.num_cores * _info.num_subcores
NPW = N // NW
C = 512
NCHUNK = NPW // C
NSL = C // 16
GB = 128
NB = 8 * C // GB

LEVELS = []
for _l in range(NLEV):
    _res = int(np.floor(BASE_RES * (SCALE ** _l)))
    LEVELS.append((_l, _res, (_res + 1) ** 3 <= T))


_mesh = plsc.VectorSubcoreMesh(core_axis_name="c", subcore_axis_name="s")


@functools.partial(
    pl.kernel,
    out_type=jax.ShapeDtypeStruct((N, OUT_D), jnp.float32),
    mesh=_mesh,
    scratch_types=[
        pltpu.VMEM((C, 3), jnp.float32),
        pltpu.VMEM((8 * C,), jnp.int32),
        pltpu.VMEM((8, C), jnp.float32),
        pltpu.VMEM((8 * C, 2), jnp.float32),
        pltpu.VMEM((C, OUT_D), jnp.float32),
        pltpu.SemaphoreType.DMA,
    ],
    compiler_params=pltpu.CompilerParams(
        needs_layout_passes=False, use_tc_tiling_on_sc=False
    ),
)
def _encode_sc(x_hbm, grid_hbm, out_hbm, x_v, idx_v, w_v, rows_v, out_v, sem):
    wid = lax.axis_index("s") * NC + lax.axis_index("c")
    iota = lax.iota(jnp.int32, 16)
    z16 = jnp.zeros((16,), jnp.int32)
    o16 = jnp.ones((16,), jnp.int32)

    def chunk_body(ci, carry):
        base = wid * NPW + ci * C
        pltpu.sync_copy(x_hbm.at[pl.ds(base, C)], x_v)

        def xcopy(s, c):
            rid = s * 16 + iota
            for d in range(3):
                cd = jnp.full((16,), d, jnp.int32)
                xd = plsc.load_gather(x_v, [rid, cd])
                plsc.store_scatter(out_v, [rid, cd], xd)
            return c

        lax.fori_loop(0, NSL, xcopy, 0)

        for (l, res, dense) in LEVELS:
            S = res + 1

            def pa(s, c, l=l, res=res, dense=dense, S=S):
                rid = s * 16 + iota
                xs = [
                    plsc.load_gather(x_v, [rid, jnp.full((16,), d, jnp.int32)])
                    for d in range(3)
                ]
                pos = [xd * jnp.float32(res) for xd in xs]
                p0 = [p.astype(jnp.int32) for p in pos]
                fr = [p - q.astype(jnp.float32) for p, q in zip(pos, p0)]
                if dense:
                    tx = [p0[0] + (l * T), p0[0] + (l * T + 1)]
                    ty = [p0[1] * S, (p0[1] + 1) * S]
                    tz = [p0[2] * (S * S), (p0[2] + 1) * (S * S)]
                else:
                    tx = [p0[0], p0[0] + 1]
                    ty = [p0[1] * P1, (p0[1] + 1) * P1]
                    tz = [p0[2] * P2, (p0[2] + 1) * P2]
                wx = [1.0 - fr[0], fr[0]]
                wy = [1.0 - fr[1], fr[1]]
                wz = [1.0 - fr[2], fr[2]]
                for corner in range(8):
                    i, j, k = corner & 1, (corner >> 1) & 1, (corner >> 2) & 1
                    if dense:
                        idx = tx[i] + ty[j] + tz[k]
                    else:
                        idx = ((tx[i] ^ ty[j] ^ tz[k]) & (T - 1)) + l * T
                    w = wx[i] * wy[j] * wz[k]
                    idx_v[pl.ds(corner * C + s * 16, 16)] = idx
                    w_v[corner, pl.ds(s * 16, 16)] = w
                return c

            lax.fori_loop(0, NSL, pa, 0)

            pltpu.async_copy(grid_hbm.at[idx_v], rows_v, sem).wait()

            col0 = 3 + 2 * l

            def pc(s, c, col0=col0):
                rid = s * 16 + iota
                acc0 = jnp.zeros((16,), jnp.float32)
                acc1 = jnp.zeros((16,), jnp.float32)
                for corner in range(8):
                    rr = corner * C + s * 16 + iota
                    g0 = plsc.load_gather(rows_v, [rr, z16])
                    g1 = plsc.load_gather(rows_v, [rr, o16])
                    w = w_v[corner, pl.ds(s * 16, 16)]
                    acc0 = acc0 + w * g0
                    acc1 = acc1 + w * g1
                plsc.store_scatter(
                    out_v, [rid, jnp.full((16,), col0, jnp.int32)], acc0
                )
                plsc.store_scatter(
                    out_v, [rid, jnp.full((16,), col0 + 1, jnp.int32)], acc1
                )
                return c

            lax.fori_loop(0, NSL, pc, 0)

        pltpu.sync_copy(out_v, out_hbm.at[pl.ds(base, C)])
        return carry

    lax.fori_loop(0, NCHUNK, chunk_body, 0)


def kernel(x, grid):
    return _encode_sc(x, grid.reshape(NLEV * T, 2))


# speedup vs baseline: 5.0877x; 5.0877x over previous
import functools

import numpy as np
import jax
import jax.numpy as jnp
from jax import lax
from jax.experimental import pallas as pl
from jax.experimental.pallas import tpu as pltpu
from jax.experimental.pallas import tpu_sc as plsc

N = 262144
NLEV = 16
T = 1 << 19
BASE_RES = 16
SCALE = 1.3819128799
P1 = np.int32(np.uint32(2654435761).astype(np.int32))
P2 = np.int32(805459861)
OUT_D = 3 + 2 * NLEV

_info = plsc.get_sparse_core_info()
NC = _info.num_cores
NW = _info.num_cores * _info.num_subcores
NPW = N // NW
C = 512
NCHUNK = NPW // C
NSL = C // 16

LEVELS = []
for _l in range(NLEV):
    _res = int(np.floor(BASE_RES * (SCALE ** _l)))
    LEVELS.append((_l, _res, (_res + 1) ** 3 <= T))


_mesh = plsc.VectorSubcoreMesh(core_axis_name="c", subcore_axis_name="s")


@functools.partial(
    pl.kernel,
    out_type=jax.ShapeDtypeStruct((OUT_D, N), jnp.float32),
    mesh=_mesh,
    scratch_types=[
        pltpu.VMEM((3, C), jnp.float32),
        pltpu.VMEM((8 * C,), jnp.int32),
        pltpu.VMEM((8 * C,), jnp.int32),
        pltpu.VMEM((8, C), jnp.float32),
        pltpu.VMEM((8 * C,), jnp.float32),
        pltpu.VMEM((8 * C,), jnp.float32),
        pltpu.VMEM((OUT_D, C), jnp.float32),
        pltpu.SemaphoreType.DMA,
    ],
    compiler_params=pltpu.CompilerParams(
        needs_layout_passes=False, use_tc_tiling_on_sc=False
    ),
)
def _encode_sc(
    xt_hbm, grid_hbm, out_hbm,
    x_v, idx0_v, idx1_v, w_v, rows0_v, rows1_v, out_v, sem,
):
    wid = lax.axis_index("s") * NC + lax.axis_index("c")
    iota = lax.iota(jnp.int32, 16)

    def chunk_body(ci, carry):
        base = wid * NPW + ci * C
        pltpu.sync_copy(xt_hbm.at[:, pl.ds(base, C)], x_v)

        def xcopy(s, c):
            for d in range(3):
                out_v[d, pl.ds(s * 16, 16)] = x_v[d, pl.ds(s * 16, 16)]
            return c

        lax.fori_loop(0, NSL, xcopy, 0)

        for (l, res, dense) in LEVELS:
            S = res + 1
            lbase = (2 * l) * T

            def pa(s, c, l=l, res=res, dense=dense, S=S, lbase=lbase):
                xs = [x_v[d, pl.ds(s * 16, 16)] for d in range(3)]
                pos = [xd * jnp.float32(res) for xd in xs]
                p0 = [p.astype(jnp.int32) for p in pos]
                fr = [p - q.astype(jnp.float32) for p, q in zip(pos, p0)]
                if dense:
                    tx = [p0[0], p0[0] + 1]
                    ty = [p0[1] * S, (p0[1] + 1) * S]
                    tz = [p0[2] * (S * S), (p0[2] + 1) * (S * S)]
                else:
                    tx = [p0[0], p0[0] + 1]
                    ty = [p0[1] * P1, (p0[1] + 1) * P1]
                    tz = [p0[2] * P2, (p0[2] + 1) * P2]
                wx = [1.0 - fr[0], fr[0]]
                wy = [1.0 - fr[1], fr[1]]
                wz = [1.0 - fr[2], fr[2]]
                for corner in range(8):
                    i, j, k = corner & 1, (corner >> 1) & 1, (corner >> 2) & 1
                    if dense:
                        loc = tx[i] + ty[j] + tz[k]
                    else:
                        loc = (tx[i] ^ ty[j] ^ tz[k]) & (T - 1)
                    e0 = loc + (loc & (-128)) + lbase
                    w = wx[i] * wy[j] * wz[k]
                    idx0_v[pl.ds(corner * C + s * 16, 16)] = e0
                    idx1_v[pl.ds(corner * C + s * 16, 16)] = e0 + 128
                    w_v[corner, pl.ds(s * 16, 16)] = w
                return c

            lax.fori_loop(0, NSL, pa, 0)

            cp0 = pltpu.async_copy(grid_hbm.at[idx0_v], rows0_v, sem)
            cp1 = pltpu.async_copy(grid_hbm.at[idx1_v], rows1_v, sem)
            cp0.wait()
            cp1.wait()

            col0 = 3 + 2 * l

            def pc(s, c, col0=col0):
                acc0 = jnp.zeros((16,), jnp.float32)
                acc1 = jnp.zeros((16,), jnp.float32)
                for corner in range(8):
                    g0 = rows0_v[pl.ds(corner * C + s * 16, 16)]
                    g1 = rows1_v[pl.ds(corner * C + s * 16, 16)]
                    w = w_v[corner, pl.ds(s * 16, 16)]
                    acc0 = acc0 + w * g0
                    acc1 = acc1 + w * g1
                out_v[col0, pl.ds(s * 16, 16)] = acc0
                out_v[col0 + 1, pl.ds(s * 16, 16)] = acc1
                return c

            lax.fori_loop(0, NSL, pc, 0)

        pltpu.sync_copy(out_v, out_hbm.at[:, pl.ds(base, C)])
        return carry

    lax.fori_loop(0, NCHUNK, chunk_body, 0)


def kernel(x, grid):
    blocked = grid.reshape(NLEV, T // 128, 128, 2).transpose(0, 1, 3, 2)
    out_t = _encode_sc(x.T, blocked.reshape(NLEV * 2 * T))
    return out_t.T


# speedup vs baseline: 7.1325x; 1.4019x over previous
import functools

import numpy as np
import jax
import jax.numpy as jnp
from jax import lax
from jax.experimental import pallas as pl
from jax.experimental.pallas import tpu as pltpu
from jax.experimental.pallas import tpu_sc as plsc

N = 262144
NLEV = 16
T = 1 << 19
BASE_RES = 16
SCALE = 1.3819128799
P1 = np.int32(np.uint32(2654435761).astype(np.int32))
P2 = np.int32(805459861)
OUT_D = 3 + 2 * NLEV

_info = plsc.get_sparse_core_info()
NC = _info.num_cores
NW = _info.num_cores * _info.num_subcores
NPW = N // NW
C = 512
NCHUNK = NPW // C
NSL = C // 16

LEVELS = []
for _l in range(NLEV):
    _res = int(np.floor(BASE_RES * (SCALE ** _l)))
    LEVELS.append((_l, _res, (_res + 1) ** 3 <= T))

NRES = 2
RES_LEVELS = [
    (l, r, -(-((r + 1) ** 3) // 128)) for (l, r, _) in LEVELS[:NRES]
]
HBM_LEVELS = LEVELS[NRES:]


_mesh = plsc.VectorSubcoreMesh(core_axis_name="c", subcore_axis_name="s")


@functools.partial(
    pl.kernel,
    out_type=jax.ShapeDtypeStruct((OUT_D, N), jnp.float32),
    mesh=_mesh,
    scratch_types=[
        pltpu.VMEM((3, C), jnp.float32),
        pltpu.VMEM((16 * C,), jnp.int32),
        pltpu.VMEM((16 * C,), jnp.int32),
        pltpu.VMEM((16, C), jnp.float32),
        pltpu.VMEM((16 * C,), jnp.float32),
        pltpu.VMEM((16 * C,), jnp.float32),
        pltpu.VMEM((OUT_D, C), jnp.float32),
        [pltpu.VMEM((nblk * 256,), jnp.float32) for (_, _, nblk) in RES_LEVELS],
        pltpu.SemaphoreType.DMA,
        pltpu.SemaphoreType.DMA,
    ],
    compiler_params=pltpu.CompilerParams(
        needs_layout_passes=False, use_tc_tiling_on_sc=False
    ),
)
def _encode_sc(
    xt_hbm, grid_hbm, out_hbm,
    x_v, idx0_v, idx1_v, w_v, rows0_v, rows1_v, out_v, tabs, sem_a, sem_b,
):
    sems = [sem_a, sem_b]
    wid = lax.axis_index("s") * NC + lax.axis_index("c")
    iota = lax.iota(jnp.int32, 16)

    for (rl, _, nblk), tab_v in zip(RES_LEVELS, tabs):
        pltpu.sync_copy(grid_hbm.at[pl.ds(rl * 2 * T, nblk * 256)], tab_v)

    def chunk_body(ci, carry):
        base = wid * NPW + ci * C
        pltpu.sync_copy(xt_hbm.at[:, pl.ds(base, C)], x_v)

        def xcopy(s, c):
            for d in range(3):
                out_v[d, pl.ds(s * 16, 16)] = x_v[d, pl.ds(s * 16, 16)]
            return c

        lax.fori_loop(0, NSL, xcopy, 0)

        for (rl, rres, _), tab_v in zip(RES_LEVELS, tabs):
            RS = rres + 1
            rcol0 = 3 + 2 * rl

            def rb(s, c, rres=rres, RS=RS, rcol0=rcol0, tab_v=tab_v):
                xs = [x_v[d, pl.ds(s * 16, 16)] for d in range(3)]
                pos = [xd * jnp.float32(rres) for xd in xs]
                p0 = [p.astype(jnp.int32) for p in pos]
                fr = [p - q.astype(jnp.float32) for p, q in zip(pos, p0)]
                tx = [p0[0], p0[0] + 1]
                ty = [p0[1] * RS, (p0[1] + 1) * RS]
                tz = [p0[2] * (RS * RS), (p0[2] + 1) * (RS * RS)]
                wx = [1.0 - fr[0], fr[0]]
                wy = [1.0 - fr[1], fr[1]]
                wz = [1.0 - fr[2], fr[2]]
                acc0 = jnp.zeros((16,), jnp.float32)
                acc1 = jnp.zeros((16,), jnp.float32)
                for corner in range(8):
                    i, j, k = corner & 1, (corner >> 1) & 1, (corner >> 2) & 1
                    loc = tx[i] + ty[j] + tz[k]
                    e0 = loc + (loc & (-128))
                    w = wx[i] * wy[j] * wz[k]
                    g0 = plsc.load_gather(tab_v, [e0])
                    g1 = plsc.load_gather(tab_v, [e0 + 128])
                    acc0 = acc0 + w * g0
                    acc1 = acc1 + w * g1
                out_v[rcol0, pl.ds(s * 16, 16)] = acc0
                out_v[rcol0 + 1, pl.ds(s * 16, 16)] = acc1
                return c

            lax.fori_loop(0, NSL, rb, 0)

        def phase_a(l, res, dense, off):
            S = res + 1
            lbase = (2 * l) * T

            def pa(s, c, l=l, res=res, dense=dense, S=S, lbase=lbase, off=off):
                xs = [x_v[d, pl.ds(s * 16, 16)] for d in range(3)]
                pos = [xd * jnp.float32(res) for xd in xs]
                p0 = [p.astype(jnp.int32) for p in pos]
                fr = [p - q.astype(jnp.float32) for p, q in zip(pos, p0)]
                if dense:
                    tx = [p0[0], p0[0] + 1]
                    ty = [p0[1] * S, (p0[1] + 1) * S]
                    tz = [p0[2] * (S * S), (p0[2] + 1) * (S * S)]
                else:
                    tx = [p0[0], p0[0] + 1]
                    ty = [p0[1] * P1, (p0[1] + 1) * P1]
                    tz = [p0[2] * P2, (p0[2] + 1) * P2]
                wx = [1.0 - fr[0], fr[0]]
                wy = [1.0 - fr[1], fr[1]]
                wz = [1.0 - fr[2], fr[2]]
                for corner in range(8):
                    i, j, k = corner & 1, (corner >> 1) & 1, (corner >> 2) & 1
                    if dense:
                        loc = tx[i] + ty[j] + tz[k]
                    else:
                        loc = (tx[i] ^ ty[j] ^ tz[k]) & (T - 1)
                    e0 = loc + (loc & (-128)) + lbase
                    w = wx[i] * wy[j] * wz[k]
                    idx0_v[pl.ds(off * 8 * C + corner * C + s * 16, 16)] = e0
                    idx1_v[pl.ds(off * 8 * C + corner * C + s * 16, 16)] = e0 + 128
                    w_v[off * 8 + corner, pl.ds(s * 16, 16)] = w
                return c

            lax.fori_loop(0, NSL, pa, 0)

        def fire(off):
            pltpu.async_copy(
                grid_hbm.at[idx0_v.at[pl.ds(off * 8 * C, 8 * C)]],
                rows0_v.at[pl.ds(off * 8 * C, 8 * C)],
                sems[off],
            )
            pltpu.async_copy(
                grid_hbm.at[idx1_v.at[pl.ds(off * 8 * C, 8 * C)]],
                rows1_v.at[pl.ds(off * 8 * C, 8 * C)],
                sems[off],
            )

        def drain(off):
            pltpu.make_async_copy(
                grid_hbm.at[idx0_v.at[pl.ds(off * 8 * C, 8 * C)]],
                rows0_v.at[pl.ds(off * 8 * C, 8 * C)],
                sems[off],
            ).wait()
            pltpu.make_async_copy(
                grid_hbm.at[idx1_v.at[pl.ds(off * 8 * C, 8 * C)]],
                rows1_v.at[pl.ds(off * 8 * C, 8 * C)],
                sems[off],
            ).wait()

        def blend(l, off):
            col0 = 3 + 2 * l

            def pc(s, c, col0=col0, off=off):
                acc0 = jnp.zeros((16,), jnp.float32)
                acc1 = jnp.zeros((16,), jnp.float32)
                for corner in range(8):
                    g0 = rows0_v[pl.ds(off * 8 * C + corner * C + s * 16, 16)]
                    g1 = rows1_v[pl.ds(off * 8 * C + corner * C + s * 16, 16)]
                    w = w_v[off * 8 + corner, pl.ds(s * 16, 16)]
                    acc0 = acc0 + w * g0
                    acc1 = acc1 + w * g1
                out_v[col0, pl.ds(s * 16, 16)] = acc0
                out_v[col0 + 1, pl.ds(s * 16, 16)] = acc1
                return c

            lax.fori_loop(0, NSL, pc, 0)

        nh = len(HBM_LEVELS)
        l0, r0, d0 = HBM_LEVELS[0]
        phase_a(l0, r0, d0, 0)
        fire(0)
        l1, r1, d1 = HBM_LEVELS[1]
        phase_a(l1, r1, d1, 1)
        fire(1)
        for t in range(2, nh):
            lt, rt, dt = HBM_LEVELS[t]
            po = t % 2
            drain(po)
            blend(HBM_LEVELS[t - 2][0], po)
            phase_a(lt, rt, dt, po)
            fire(po)
        drain(nh % 2)
        blend(HBM_LEVELS[nh - 2][0], nh % 2)
        drain((nh - 1) % 2)
        blend(HBM_LEVELS[nh - 1][0], (nh - 1) % 2)

        pltpu.sync_copy(out_v, out_hbm.at[:, pl.ds(base, C)])
        return carry

    lax.fori_loop(0, NCHUNK, chunk_body, 0)


def kernel(x, grid):
    blocked = grid.reshape(NLEV, T // 128, 128, 2).transpose(0, 1, 3, 2)
    out_t = _encode_sc(x.T, blocked.reshape(NLEV * 2 * T))
    return out_t.T


# speedup vs baseline: 8.9606x; 1.2563x over previous
import functools

import numpy as np
import jax
import jax.numpy as jnp
from jax import lax
from jax.experimental import pallas as pl
from jax.experimental.pallas import tpu as pltpu
from jax.experimental.pallas import tpu_sc as plsc

N = 262144
NLEV = 16
T = 1 << 19
BASE_RES = 16
SCALE = 1.3819128799
P1 = np.int32(np.uint32(2654435761).astype(np.int32))
P2 = np.int32(805459861)
OUT_D = 3 + 2 * NLEV

_info = plsc.get_sparse_core_info()
NC = _info.num_cores
NS = _info.num_subcores
NW = NC * NS
NPW = N // NW
C = 256
NCHUNK = NPW // C
NSL = C // 16

NBLK = NLEV * (T // 128)
BPT = NBLK // NS
NBAT = 16

LEVELS = []
for _l in range(NLEV):
    _res = int(np.floor(BASE_RES * (SCALE ** _l)))
    LEVELS.append((_l, _res, (_res + 1) ** 3 <= T))

NRES = 2
RES_LEVELS = [(l, r, -(-((r + 1) ** 3) // 128)) for (l, r, _) in LEVELS[:NRES]]
HBM_LEVELS = LEVELS[NRES:]


_mesh = plsc.VectorSubcoreMesh(core_axis_name="c", subcore_axis_name="s")


@functools.partial(
    pl.kernel,
    out_type=jax.ShapeDtypeStruct((OUT_D, N), jnp.float32),
    mesh=_mesh,
    scratch_types=[
        pltpu.HBM((2 * NLEV * T, 2), jnp.float32),
        pltpu.VMEM((3, C), jnp.float32),
        pltpu.VMEM((16 * C,), jnp.int32),
        pltpu.VMEM((16, C), jnp.float32),
        pltpu.VMEM((16 * C, 2), jnp.float32),
        pltpu.VMEM((OUT_D, C), jnp.float32),
        pltpu.VMEM((NBAT * 256,), jnp.float32),
        pltpu.VMEM((NBAT * 128, 2), jnp.float32),
        [pltpu.VMEM((nblk * 256,), jnp.float32) for (_, _, nblk) in RES_LEVELS],
        pltpu.SemaphoreType.DMA,
        pltpu.SemaphoreType.DMA,
    ],
    compiler_params=pltpu.CompilerParams(
        needs_layout_passes=False, use_tc_tiling_on_sc=False
    ),
)
def _encode_sc(
    xt_hbm, gnat_hbm, out_hbm,
    tbl_hbm, x_v, idx_v, w_v, rows_v, out_v, cin_v, cout_v, tabs,
    sem_a, sem_b,
):
    sems = [sem_a, sem_b]
    core = lax.axis_index("c")
    sub = lax.axis_index("s")
    wid = sub * NC + core
    iota = lax.iota(jnp.int32, 16)
    z16 = jnp.zeros((16,), jnp.int32)
    o16 = jnp.ones((16,), jnp.int32)
    cbase = core * (NLEV * T)

    def conv(it, carry):
        q0 = sub * BPT + it * NBAT
        pltpu.sync_copy(gnat_hbm.at[pl.ds(q0 * 256, NBAT * 256)], cin_v)

        def shuf(ms, c):
            q = ms // 8
            m = (ms % 8) * 16
            rid = q * 128 + m + iota
            v0 = cin_v[pl.ds(q * 256 + m, 16)]
            v1 = cin_v[pl.ds(q * 256 + 128 + m, 16)]
            plsc.store_scatter(cout_v, [rid, z16], v0)
            plsc.store_scatter(cout_v, [rid, o16], v1)
            return c

        lax.fori_loop(0, NBAT * 8, shuf, 0)
        pltpu.sync_copy(
            cout_v, tbl_hbm.at[pl.ds(cbase + q0 * 128, NBAT * 128)]
        )
        return carry

    lax.fori_loop(0, BPT // NBAT, conv, 0)

    for (rl, _, nblk), tab_v in zip(RES_LEVELS, tabs):
        pltpu.sync_copy(gnat_hbm.at[pl.ds(rl * 2 * T, nblk * 256)], tab_v)

    plsc.subcore_barrier()

    def chunk_body(ci, carry):
        base = wid * NPW + ci * C
        pltpu.sync_copy(xt_hbm.at[:, pl.ds(base, C)], x_v)

        def xcopy(s, c):
            for d in range(3):
                out_v[d, pl.ds(s * 16, 16)] = x_v[d, pl.ds(s * 16, 16)]
            return c

        lax.fori_loop(0, NSL, xcopy, 0)

        for (rl, rres, _), tab_v in zip(RES_LEVELS, tabs):
            RS = rres + 1
            rcol0 = 3 + 2 * rl

            def rb(s, c, rres=rres, RS=RS, rcol0=rcol0, tab_v=tab_v):
                xs = [x_v[d, pl.ds(s * 16, 16)] for d in range(3)]
                pos = [xd * jnp.float32(rres) for xd in xs]
                p0 = [p.astype(jnp.int32) for p in pos]
                fr = [p - q.astype(jnp.float32) for p, q in zip(pos, p0)]
                tx = [p0[0], p0[0] + 1]
                ty = [p0[1] * RS, (p0[1] + 1) * RS]
                tz = [p0[2] * (RS * RS), (p0[2] + 1) * (RS * RS)]
                wx = [1.0 - fr[0], fr[0]]
                wy = [1.0 - fr[1], fr[1]]
                wz = [1.0 - fr[2], fr[2]]
                acc0 = jnp.zeros((16,), jnp.float32)
                acc1 = jnp.zeros((16,), jnp.float32)
                for corner in range(8):
                    i, j, k = corner & 1, (corner >> 1) & 1, (corner >> 2) & 1
                    loc = tx[i] + ty[j] + tz[k]
                    e0 = loc + (loc & (-128))
                    w = wx[i] * wy[j] * wz[k]
                    g0 = plsc.load_gather(tab_v, [e0])
                    g1 = plsc.load_gather(tab_v, [e0 + 128])
                    acc0 = acc0 + w * g0
                    acc1 = acc1 + w * g1
                out_v[rcol0, pl.ds(s * 16, 16)] = acc0
                out_v[rcol0 + 1, pl.ds(s * 16, 16)] = acc1
                return c

            lax.fori_loop(0, NSL, rb, 0)

        def phase_a(l, res, dense, off):
            S = res + 1
            lbase = cbase + l * T

            def pa(s, c, res=res, dense=dense, S=S, lbase=lbase, off=off):
                xs = [x_v[d, pl.ds(s * 16, 16)] for d in range(3)]
                pos = [xd * jnp.float32(res) for xd in xs]
                p0 = [p.astype(jnp.int32) for p in pos]
                fr = [p - q.astype(jnp.float32) for p, q in zip(pos, p0)]
                if dense:
                    tx = [p0[0] + lbase, p0[0] + lbase + 1]
                    ty = [p0[1] * S, (p0[1] + 1) * S]
                    tz = [p0[2] * (S * S), (p0[2] + 1) * (S * S)]
                else:
                    tx = [p0[0], p0[0] + 1]
                    ty = [p0[1] * P1, (p0[1] + 1) * P1]
                    tz = [p0[2] * P2, (p0[2] + 1) * P2]
                wx = [1.0 - fr[0], fr[0]]
                wy = [1.0 - fr[1], fr[1]]
                wz = [1.0 - fr[2], fr[2]]
                for corner in range(8):
                    i, j, k = corner & 1, (corner >> 1) & 1, (corner >> 2) & 1
                    if dense:
                        idx = tx[i] + ty[j] + tz[k]
                    else:
                        idx = ((tx[i] ^ ty[j] ^ tz[k]) & (T - 1)) + lbase
                    w = wx[i] * wy[j] * wz[k]
                    idx_v[pl.ds(off * 8 * C + corner * C + s * 16, 16)] = idx
                    w_v[off * 8 + corner, pl.ds(s * 16, 16)] = w
                return c

            lax.fori_loop(0, NSL, pa, 0)

        def fire(off):
            pltpu.async_copy(
                tbl_hbm.at[idx_v.at[pl.ds(off * 8 * C, 8 * C)]],
                rows_v.at[pl.ds(off * 8 * C, 8 * C)],
                sems[off],
            )

        def drain(off):
            pltpu.make_async_copy(
                tbl_hbm.at[idx_v.at[pl.ds(off * 8 * C, 8 * C)]],
                rows_v.at[pl.ds(off * 8 * C, 8 * C)],
                sems[off],
            ).wait()

        def blend(l, off):
            col0 = 3 + 2 * l

            def pc(s, c, col0=col0, off=off):
                acc0 = jnp.zeros((16,), jnp.float32)
                acc1 = jnp.zeros((16,), jnp.float32)
                for corner in range(8):
                    rr = off * 8 * C + corner * C + s * 16 + iota
                    g0 = plsc.load_gather(rows_v, [rr, z16])
                    g1 = plsc.load_gather(rows_v, [rr, o16])
                    w = w_v[off * 8 + corner, pl.ds(s * 16, 16)]
                    acc0 = acc0 + w * g0
                    acc1 = acc1 + w * g1
                out_v[col0, pl.ds(s * 16, 16)] = acc0
                out_v[col0 + 1, pl.ds(s * 16, 16)] = acc1
                return c

            lax.fori_loop(0, NSL, pc, 0)

        nh = len(HBM_LEVELS)
        l0, r0, d0 = HBM_LEVELS[0]
        phase_a(l0, r0, d0, 0)
        fire(0)
        l1, r1, d1 = HBM_LEVELS[1]
        phase_a(l1, r1, d1, 1)
        fire(1)
        for t in range(2, nh):
            lt, rt, dt = HBM_LEVELS[t]
            po = t % 2
            drain(po)
            blend(HBM_LEVELS[t - 2][0], po)
            phase_a(lt, rt, dt, po)
            fire(po)
        drain(nh % 2)
        blend(HBM_LEVELS[nh - 2][0], nh % 2)
        drain((nh - 1) % 2)
        blend(HBM_LEVELS[nh - 1][0], (nh - 1) % 2)

        pltpu.sync_copy(out_v, out_hbm.at[:, pl.ds(base, C)])
        return carry

    lax.fori_loop(0, NCHUNK, chunk_body, 0)


def kernel(x, grid):
    gnat = (
        grid.reshape(NLEV, T // 128, 128, 2)
        .transpose(0, 1, 3, 2)
        .reshape(NLEV * 2 * T)
    )
    out_t = _encode_sc(x.T, gnat)
    return out_t.T


# speedup vs baseline: 10.5871x; 1.1815x over previous
import functools

import numpy as np
import jax
import jax.numpy as jnp
from jax import lax
from jax.experimental import pallas as pl
from jax.experimental.pallas import tpu as pltpu
from jax.experimental.pallas import tpu_sc as plsc

N = 262144
NLEV = 16
T = 1 << 19
BASE_RES = 16
SCALE = 1.3819128799
P1 = np.int32(np.uint32(2654435761).astype(np.int32))
P2 = np.int32(805459861)
OUT_D = 3 + 2 * NLEV

_info = plsc.get_sparse_core_info()
NC = _info.num_cores
NS = _info.num_subcores
NW = NC * NS
NPW = N // NW
C = 256
NCHUNK = NPW // C
NSL = C // 16

NBLK = NLEV * (T // 128)
BPT = NBLK // NW
NBAT = 16

LEVELS = []
for _l in range(NLEV):
    _res = int(np.floor(BASE_RES * (SCALE ** _l)))
    LEVELS.append((_l, _res, (_res + 1) ** 3 <= T))

NRES = 2
RES_LEVELS = [(l, r, -(-((r + 1) ** 3) // 128)) for (l, r, _) in LEVELS[:NRES]]
HBM_LEVELS = LEVELS[NRES:]


_mesh = plsc.VectorSubcoreMesh(core_axis_name="c", subcore_axis_name="s")


@functools.partial(
    pl.kernel,
    out_type=jax.ShapeDtypeStruct((OUT_D, N), jnp.float32),
    mesh=_mesh,
    scratch_types=[
        pltpu.VMEM((3, C), jnp.float32),
        pltpu.VMEM((16 * C,), jnp.int32),
        pltpu.VMEM((16, C), jnp.float32),
        pltpu.VMEM((16 * C, 8), jnp.float32),
        pltpu.VMEM((OUT_D, C), jnp.float32),
        [pltpu.VMEM((nblk * 256,), jnp.float32) for (_, _, nblk) in RES_LEVELS],
        pltpu.SemaphoreType.DMA,
        pltpu.SemaphoreType.DMA,
    ],
    compiler_params=pltpu.CompilerParams(
        needs_layout_passes=False, use_tc_tiling_on_sc=False
    ),
)
def _encode_sc(
    xt_hbm, gnat_hbm, tbl_hbm, out_hbm,
    x_v, idx_v, w_v, rows_v, out_v, tabs,
    sem_a, sem_b,
):
    sems = [sem_a, sem_b]
    wid = lax.axis_index("s") * NC + lax.axis_index("c")
    iota = lax.iota(jnp.int32, 16)
    z16 = jnp.zeros((16,), jnp.int32)
    o16 = jnp.ones((16,), jnp.int32)
    cbase = 0

    for (rl, _, nblk), tab_v in zip(RES_LEVELS, tabs):
        pltpu.sync_copy(gnat_hbm.at[pl.ds(rl * 2 * T, nblk * 256)], tab_v)

    def chunk_body(ci, carry):
        base = wid * NPW + ci * C
        pltpu.sync_copy(xt_hbm.at[:, pl.ds(base, C)], x_v)

        def xcopy(s, c):
            for d in range(3):
                out_v[d, pl.ds(s * 16, 16)] = x_v[d, pl.ds(s * 16, 16)]
            return c

        lax.fori_loop(0, NSL, xcopy, 0)

        for (rl, rres, _), tab_v in zip(RES_LEVELS, tabs):
            RS = rres + 1
            rcol0 = 3 + 2 * rl

            def rb(s, c, rres=rres, RS=RS, rcol0=rcol0, tab_v=tab_v):
                xs = [x_v[d, pl.ds(s * 16, 16)] for d in range(3)]
                pos = [xd * jnp.float32(rres) for xd in xs]
                p0 = [p.astype(jnp.int32) for p in pos]
                fr = [p - q.astype(jnp.float32) for p, q in zip(pos, p0)]
                tx = [p0[0], p0[0] + 1]
                ty = [p0[1] * RS, (p0[1] + 1) * RS]
                tz = [p0[2] * (RS * RS), (p0[2] + 1) * (RS * RS)]
                wx = [1.0 - fr[0], fr[0]]
                wy = [1.0 - fr[1], fr[1]]
                wz = [1.0 - fr[2], fr[2]]
                acc0 = jnp.zeros((16,), jnp.float32)
                acc1 = jnp.zeros((16,), jnp.float32)
                for corner in range(8):
                    i, j, k = corner & 1, (corner >> 1) & 1, (corner >> 2) & 1
                    loc = tx[i] + ty[j] + tz[k]
                    e0 = loc + (loc & (-128))
                    w = wx[i] * wy[j] * wz[k]
                    g0 = plsc.load_gather(tab_v, [e0])
                    g1 = plsc.load_gather(tab_v, [e0 + 128])
                    acc0 = acc0 + w * g0
                    acc1 = acc1 + w * g1
                out_v[rcol0, pl.ds(s * 16, 16)] = acc0
                out_v[rcol0 + 1, pl.ds(s * 16, 16)] = acc1
                return c

            lax.fori_loop(0, NSL, rb, 0)

        def phase_a(l, res, dense, off):
            S = res + 1
            lbase = cbase + l * T

            def pa(s, c, res=res, dense=dense, S=S, lbase=lbase, off=off):
                xs = [x_v[d, pl.ds(s * 16, 16)] for d in range(3)]
                pos = [xd * jnp.float32(res) for xd in xs]
                p0 = [p.astype(jnp.int32) for p in pos]
                fr = [p - q.astype(jnp.float32) for p, q in zip(pos, p0)]
                if dense:
                    tx = [p0[0] + lbase, p0[0] + lbase + 1]
                    ty = [p0[1] * S, (p0[1] + 1) * S]
                    tz = [p0[2] * (S * S), (p0[2] + 1) * (S * S)]
                else:
                    tx = [p0[0], p0[0] + 1]
                    ty = [p0[1] * P1, (p0[1] + 1) * P1]
                    tz = [p0[2] * P2, (p0[2] + 1) * P2]
                wx = [1.0 - fr[0], fr[0]]
                wy = [1.0 - fr[1], fr[1]]
                wz = [1.0 - fr[2], fr[2]]
                for corner in range(8):
                    i, j, k = corner & 1, (corner >> 1) & 1, (corner >> 2) & 1
                    if dense:
                        idx = tx[i] + ty[j] + tz[k]
                    else:
                        idx = ((tx[i] ^ ty[j] ^ tz[k]) & (T - 1)) + lbase
                    w = wx[i] * wy[j] * wz[k]
                    idx_v[pl.ds(off * 8 * C + corner * C + s * 16, 16)] = idx
                    w_v[off * 8 + corner, pl.ds(s * 16, 16)] = w
                return c

            lax.fori_loop(0, NSL, pa, 0)

        def fire(off):
            pltpu.async_copy(
                tbl_hbm.at[idx_v.at[pl.ds(off * 8 * C, 8 * C)]],
                rows_v.at[pl.ds(off * 8 * C, 8 * C)],
                sems[off],
            )

        def drain(off):
            pltpu.make_async_copy(
                tbl_hbm.at[idx_v.at[pl.ds(off * 8 * C, 8 * C)]],
                rows_v.at[pl.ds(off * 8 * C, 8 * C)],
                sems[off],
            ).wait()

        def blend(l, off):
            col0 = 3 + 2 * l

            def pc(s, c, col0=col0, off=off):
                acc0 = jnp.zeros((16,), jnp.float32)
                acc1 = jnp.zeros((16,), jnp.float32)
                for corner in range(8):
                    rr = off * 8 * C + corner * C + s * 16 + iota
                    g0 = plsc.load_gather(rows_v, [rr, z16])
                    g1 = plsc.load_gather(rows_v, [rr, o16])
                    w = w_v[off * 8 + corner, pl.ds(s * 16, 16)]
                    acc0 = acc0 + w * g0
                    acc1 = acc1 + w * g1
                out_v[col0, pl.ds(s * 16, 16)] = acc0
                out_v[col0 + 1, pl.ds(s * 16, 16)] = acc1
                return c

            lax.fori_loop(0, NSL, pc, 0)

        nh = len(HBM_LEVELS)
        l0, r0, d0 = HBM_LEVELS[0]
        phase_a(l0, r0, d0, 0)
        fire(0)
        l1, r1, d1 = HBM_LEVELS[1]
        phase_a(l1, r1, d1, 1)
        fire(1)
        for t in range(2, nh):
            lt, rt, dt = HBM_LEVELS[t]
            po = t % 2
            drain(po)
            blend(HBM_LEVELS[t - 2][0], po)
            phase_a(lt, rt, dt, po)
            fire(po)
        drain(nh % 2)
        blend(HBM_LEVELS[nh - 2][0], nh % 2)
        drain((nh - 1) % 2)
        blend(HBM_LEVELS[nh - 1][0], (nh - 1) % 2)

        pltpu.sync_copy(out_v, out_hbm.at[:, pl.ds(base, C)])
        return carry

    lax.fori_loop(0, NCHUNK, chunk_body, 0)


@functools.partial(
    pl.kernel,
    out_type=jax.ShapeDtypeStruct((NLEV * T, 8), jnp.float32),
    mesh=_mesh,
    scratch_types=[
        pltpu.VMEM((NBAT * 256,), jnp.float32),
        pltpu.VMEM((NBAT * 128, 8), jnp.float32),
    ],
    compiler_params=pltpu.CompilerParams(
        needs_layout_passes=False, use_tc_tiling_on_sc=False
    ),
)
def _convert_sc(gnat_hbm, tbl_hbm, cin_v, cout_v):
    wid = lax.axis_index("s") * NC + lax.axis_index("c")
    iota = lax.iota(jnp.int32, 16)
    z16 = jnp.zeros((16,), jnp.int32)
    o16 = jnp.ones((16,), jnp.int32)

    def conv(it, carry):
        q0 = wid * BPT + it * NBAT
        pltpu.sync_copy(gnat_hbm.at[pl.ds(q0 * 256, NBAT * 256)], cin_v)

        def shuf(ms, c):
            q = ms // 8
            m = (ms % 8) * 16
            rid = q * 128 + m + iota
            v0 = cin_v[pl.ds(q * 256 + m, 16)]
            v1 = cin_v[pl.ds(q * 256 + 128 + m, 16)]
            plsc.store_scatter(cout_v, [rid, z16], v0)
            plsc.store_scatter(cout_v, [rid, o16], v1)
            return c

        lax.fori_loop(0, NBAT * 8, shuf, 0)
        pltpu.sync_copy(cout_v, tbl_hbm.at[pl.ds(q0 * 128, NBAT * 128)])
        return carry

    lax.fori_loop(0, BPT // NBAT, conv, 0)


def kernel(x, grid):
    gnat = (
        grid.reshape(NLEV, T // 128, 128, 2)
        .transpose(0, 1, 3, 2)
        .reshape(NLEV * 2 * T)
    )
    table = _convert_sc(gnat)
    out_t = _encode_sc(x.T, gnat, table)
    return out_t.T
